# Initial kernel scaffold; baseline (speedup 1.0000x reference)
import jax
import jax.numpy as jnp
from jax import lax
from jax.experimental import pallas as pl
from jax.experimental.pallas import tpu as pltpu
from jax.experimental.pallas import tpu_sc as plsc

NC = 2
NS = 16
NW = NC * NS
C = 128
CW = 16
ZR = 125


def _make_segsum(n, d, nch, with_cnt):
    epw = nch * C
    rpt = n // NS
    np1 = n + 1
    mesh = plsc.VectorSubcoreMesh(core_axis_name="c", subcore_axis_name="s",
                                  num_cores=NC, num_subcores=NS)
    out_type = [jax.ShapeDtypeStruct((NC, n, d), jnp.float32)]
    scratch = [
        pltpu.VMEM_SHARED((np1, d), jnp.float32),
        pltpu.VMEM((epw,), jnp.int32),
        pltpu.VMEM((C,), jnp.int32),
        pltpu.VMEM((C, d), jnp.float32),
        pltpu.VMEM((ZR, d), jnp.float32),
        pltpu.SemaphoreType.DMA,
    ]
    if with_cnt:
        out_type.append(jax.ShapeDtypeStruct((NC, n, CW), jnp.float32))
        scratch += [
            pltpu.VMEM_SHARED((np1, CW), jnp.float32),
            pltpu.VMEM((C, CW), jnp.float32),
            pltpu.VMEM((rpt, CW), jnp.float32),
        ]

    def body(vals_hbm, src_hbm, dst_hbm, *rest):
        if with_cnt:
            (out_hbm, outc_hbm, acc_sh, src_v, dstc_v, rows_v, zbuf, gsem,
             cnt_sh, ones_v, zcnt) = rest
        else:
            out_hbm, acc_sh, src_v, dstc_v, rows_v, zbuf, gsem = rest
        cid = lax.axis_index("c")
        sid = lax.axis_index("s")
        wid = sid * NC + cid

        z16 = jnp.zeros((16,), jnp.float32)
        dl = d // 16

        def zb(i, _):
            zbuf[i // dl, pl.ds((i % dl) * 16, 16)] = z16
            return 0
        lax.fori_loop(0, ZR * dl, zb, 0)
        for t in range(rpt // ZR):
            pltpu.sync_copy(zbuf, acc_sh.at[pl.ds(sid * rpt + t * ZR, ZR)])
        if with_cnt:
            def zc(i, _):
                zcnt[i, pl.ds(0, 16)] = z16
                return 0
            lax.fori_loop(0, rpt, zc, 0)
            o16 = jnp.ones((16,), jnp.float32)

            def ob(i, _):
                ones_v[i, pl.ds(0, 16)] = o16
                return 0
            lax.fori_loop(0, C, ob, 0)
            pltpu.sync_copy(zcnt, cnt_sh.at[pl.ds(sid * rpt, rpt)])

        @pl.when(sid == NS - 1)
        def _():
            pltpu.sync_copy(zbuf.at[pl.ds(0, 1)], acc_sh.at[pl.ds(n, 1)])
            if with_cnt:
                pltpu.sync_copy(zcnt.at[pl.ds(0, 1)], cnt_sh.at[pl.ds(n, 1)])

        plsc.subcore_barrier()

        pltpu.sync_copy(src_hbm.at[pl.ds(wid * epw, epw)], src_v)

        def step(j, _):
            pltpu.sync_copy(dst_hbm.at[wid * nch + j], dstc_v)
            pltpu.async_copy(
                vals_hbm.at[src_v.at[pl.ds(j * C, C)]], rows_v, gsem).wait()
            pltpu.sync_copy(rows_v, acc_sh.at[dstc_v], add=True)
            if with_cnt:
                pltpu.sync_copy(ones_v, cnt_sh.at[dstc_v], add=True)
            return 0
        lax.fori_loop(0, nch, step, 0)

        plsc.subcore_barrier()

        pltpu.sync_copy(acc_sh.at[pl.ds(sid * rpt, rpt)],
                        out_hbm.at[cid, pl.ds(sid * rpt, rpt)])
        if with_cnt:
            pltpu.sync_copy(cnt_sh.at[pl.ds(sid * rpt, rpt)],
                            outc_hbm.at[cid, pl.ds(sid * rpt, rpt)])

    return pl.kernel(body, out_type=out_type, mesh=mesh,
                     scratch_types=scratch)


def _tc1_body(s1p, cntp, x, w1lT, b1, w1rT, w2lT, h_out, g_out):
    s1 = s1p[0] + s1p[1]
    c = cntp[0] + cntp[1]
    inv = 1.0 / jnp.maximum(c[:, 0:1], 1.0)
    t = (jnp.dot(s1 * inv, w1lT[...], preferred_element_type=jnp.float32)
         + jnp.dot(x[...], w1rT[...], preferred_element_type=jnp.float32)
         + b1[...])
    h = jnp.maximum(t, 0.0)
    h_out[...] = h
    g_out[...] = jnp.dot(h, w2lT[...], preferred_element_type=jnp.float32)


def _tc2_body(s2p, cntp, h, w2rT, b2, out):
    s2 = s2p[0] + s2p[1]
    c = cntp[0] + cntp[1]
    inv = 1.0 / jnp.maximum(c[:, 0:1], 1.0)
    out[...] = (s2 * inv
                + jnp.dot(h[...], w2rT[...], preferred_element_type=jnp.float32)
                + b2[...])


def kernel(x, ei, W1_l, b1_l, W1_r, W2_l, b2_l, W2_r):
    n, d_in = x.shape
    e = ei.shape[1]
    hid = W1_l.shape[0]
    d_out = W2_l.shape[0]

    epc = NW * C
    nch = -(-e // epc)
    pad = nch * epc - e
    src = jnp.concatenate([ei[0], jnp.zeros((pad,), jnp.int32)])
    dst = jnp.concatenate([ei[1], jnp.full((pad,), n, jnp.int32)])
    dst2 = dst.reshape(NW * nch, C)

    s1p, cntp = _make_segsum(n, d_in, nch, True)(x, src, dst2)

    blk = 2500
    grid = (n // blk,)
    full = lambda shape: pl.BlockSpec(shape, lambda i: tuple(0 for _ in shape))
    rows3 = lambda w: pl.BlockSpec((NC, blk, w), lambda i: (0, i, 0))
    rows2 = lambda w: pl.BlockSpec((blk, w), lambda i: (i, 0))

    h, g = pl.pallas_call(
        _tc1_body,
        grid=grid,
        in_specs=[rows3(d_in), rows3(CW), rows2(d_in),
                  full((d_in, hid)), full((1, hid)), full((d_in, hid)),
                  full((hid, d_out))],
        out_specs=[rows2(hid), rows2(d_out)],
        out_shape=[jax.ShapeDtypeStruct((n, hid), jnp.float32),
                   jax.ShapeDtypeStruct((n, d_out), jnp.float32)],
    )(s1p, cntp, x, W1_l.T, b1_l.reshape(1, -1), W1_r.T, W2_l.T)

    s2p, = _make_segsum(n, d_out, nch, False)(g, src, dst2)

    out = pl.pallas_call(
        _tc2_body,
        grid=grid,
        in_specs=[rows3(d_out), rows3(CW), rows2(hid),
                  full((hid, d_out)), full((1, d_out))],
        out_specs=rows2(d_out),
        out_shape=jax.ShapeDtypeStruct((n, d_out), jnp.float32),
    )(s2p, cntp, h, W2_r.T, b2_l.reshape(1, -1))
    return out


# speedup vs baseline: 4.9923x; 4.9923x over previous
import jax
import jax.numpy as jnp
from jax import lax
from jax.experimental import pallas as pl
from jax.experimental.pallas import tpu as pltpu
from jax.experimental.pallas import tpu_sc as plsc

NC = 2
NS = 16
NW = NC * NS
C = 128
CW = 16
ZR = 24


def _make_segsum(n, d, nch):
    epw = nch * C
    rw = (n // NS) // 8 * 8
    rem = n - NS * rw
    np1 = n + 1
    mesh = plsc.VectorSubcoreMesh(core_axis_name="c", subcore_axis_name="s",
                                  num_cores=NC, num_subcores=NS)
    scratch = [
        pltpu.VMEM_SHARED((np1, d), jnp.float32),
        pltpu.VMEM((epw,), jnp.int32),
        pltpu.VMEM((C,), jnp.int32),
        pltpu.VMEM((C, d), jnp.float32),
        pltpu.VMEM((ZR, d), jnp.float32),
        pltpu.SemaphoreType.DMA,
    ]

    def body(vals_hbm, src_hbm, dst_hbm, out_hbm,
             acc_sh, src_v, dstc_v, rows_v, zbuf, gsem):
        cid = lax.axis_index("c")
        sid = lax.axis_index("s")
        wid = sid * NC + cid

        z16 = jnp.zeros((16,), jnp.float32)
        dl = d // 16

        def zb(i, _):
            zbuf[i // dl, pl.ds((i % dl) * 16, 16)] = z16
            return 0
        lax.fori_loop(0, ZR * dl, zb, 0)
        for t in range(rw // ZR):
            pltpu.sync_copy(zbuf, acc_sh.at[pl.ds(sid * rw + t * ZR, ZR)])

        @pl.when(sid == NS - 1)
        def _():
            pltpu.sync_copy(zbuf.at[pl.ds(0, rem + 1)],
                            acc_sh.at[pl.ds(NS * rw, rem + 1)])

        plsc.subcore_barrier()

        pltpu.sync_copy(src_hbm.at[pl.ds(wid * epw, epw)], src_v)

        def step(j, _):
            pltpu.sync_copy(dst_hbm.at[pl.ds(wid * epw + j * C, C)], dstc_v)
            pltpu.async_copy(
                vals_hbm.at[src_v.at[pl.ds(j * C, C)]], rows_v, gsem).wait()
            pltpu.sync_copy(rows_v, acc_sh.at[dstc_v], add=True)
            return 0
        lax.fori_loop(0, nch, step, 0)

        plsc.subcore_barrier()

        pltpu.sync_copy(acc_sh.at[pl.ds(sid * rw, rw)],
                        out_hbm.at[cid, pl.ds(sid * rw, rw)])

        @pl.when(sid == NS - 1)
        def _():
            pltpu.sync_copy(acc_sh.at[pl.ds(NS * rw, rem)],
                            out_hbm.at[cid, pl.ds(NS * rw, rem)])

    return pl.kernel(body,
                     out_type=jax.ShapeDtypeStruct((NC, n, d), jnp.float32),
                     mesh=mesh, scratch_types=scratch)


def _make_cnt(n, d, nch):
    epw = nch * C
    rw = (n // NS) // 8 * 8
    rem = n - NS * rw
    np1 = n + 1
    mesh = plsc.VectorSubcoreMesh(core_axis_name="c", subcore_axis_name="s",
                                  num_cores=NC, num_subcores=NS)
    scratch = [
        pltpu.VMEM_SHARED((np1, d), jnp.float32),
        pltpu.VMEM((C,), jnp.int32),
        pltpu.VMEM((C, d), jnp.float32),
        pltpu.VMEM((ZR, d), jnp.float32),
    ]

    def body(dst_hbm, outc_hbm, cnt_sh, dstc_v, ones_v, zcnt):
        cid = lax.axis_index("c")
        sid = lax.axis_index("s")
        wid = sid * NC + cid

        z16 = jnp.zeros((16,), jnp.float32)
        o16 = jnp.ones((16,), jnp.float32)
        dl = d // 16

        def zc(i, _):
            zcnt[i // dl, pl.ds((i % dl) * 16, 16)] = z16
            return 0
        lax.fori_loop(0, ZR * dl, zc, 0)

        def ob(i, _):
            ones_v[i // dl, pl.ds((i % dl) * 16, 16)] = o16
            return 0
        lax.fori_loop(0, C * dl, ob, 0)
        for t in range(rw // ZR):
            pltpu.sync_copy(zcnt, cnt_sh.at[pl.ds(sid * rw + t * ZR, ZR)])

        @pl.when(sid == NS - 1)
        def _():
            pltpu.sync_copy(zcnt.at[pl.ds(0, rem + 1)],
                            cnt_sh.at[pl.ds(NS * rw, rem + 1)])

        plsc.subcore_barrier()

        def step(j, _):
            pltpu.sync_copy(dst_hbm.at[pl.ds(wid * epw + j * C, C)], dstc_v)
            pltpu.sync_copy(ones_v, cnt_sh.at[dstc_v], add=True)
            return 0
        lax.fori_loop(0, nch, step, 0)

        plsc.subcore_barrier()

        pltpu.sync_copy(cnt_sh.at[pl.ds(sid * rw, rw)],
                        outc_hbm.at[cid, pl.ds(sid * rw, rw)])

        @pl.when(sid == NS - 1)
        def _():
            pltpu.sync_copy(cnt_sh.at[pl.ds(NS * rw, rem)],
                            outc_hbm.at[cid, pl.ds(NS * rw, rem)])

    return pl.kernel(body,
                     out_type=jax.ShapeDtypeStruct((NC, n, d), jnp.float32),
                     mesh=mesh, scratch_types=scratch)


def _tc1_body(s1p, cntp, x, w1lT, b1, w1rT, w2lT, h_out, g_out):
    s1 = s1p[0] + s1p[1]
    c = cntp[0] + cntp[1]
    inv = 1.0 / jnp.maximum(c[:, 0:1], 1.0)
    t = (jnp.dot(s1 * inv, w1lT[...], preferred_element_type=jnp.float32)
         + jnp.dot(x[...], w1rT[...], preferred_element_type=jnp.float32)
         + b1[...])
    h = jnp.maximum(t, 0.0)
    h_out[...] = h
    g_out[...] = jnp.dot(h, w2lT[...], preferred_element_type=jnp.float32)


def _tc2_body(s2p, cntp, h, w2rT, b2, out):
    s2 = s2p[0] + s2p[1]
    c = cntp[0] + cntp[1]
    inv = 1.0 / jnp.maximum(c[:, 0:1], 1.0)
    out[...] = (s2 * inv
                + jnp.dot(h[...], w2rT[...], preferred_element_type=jnp.float32)
                + b2[...])


def kernel(x, ei, W1_l, b1_l, W1_r, W2_l, b2_l, W2_r):
    n, d_in = x.shape
    e = ei.shape[1]
    hid = W1_l.shape[0]
    d_out = W2_l.shape[0]

    epc = NW * C
    nch = -(-e // epc)
    pad = nch * epc - e
    src = jnp.concatenate([ei[0], jnp.zeros((pad,), jnp.int32)])
    dst = jnp.concatenate([ei[1], jnp.full((pad,), n, jnp.int32)])

    cntp = _make_cnt(n, d_in, nch)(dst)
    s1p = _make_segsum(n, d_in, nch)(x, src, dst)

    blk = 2000
    grid = (n // blk,)
    full = lambda shape: pl.BlockSpec(shape, lambda i: tuple(0 for _ in shape))
    rows3 = lambda w: pl.BlockSpec((NC, blk, w), lambda i: (0, i, 0))
    rows2 = lambda w: pl.BlockSpec((blk, w), lambda i: (i, 0))

    h, g = pl.pallas_call(
        _tc1_body,
        grid=grid,
        in_specs=[rows3(d_in), rows3(d_in), rows2(d_in),
                  full((d_in, hid)), full((1, hid)), full((d_in, hid)),
                  full((hid, d_out))],
        out_specs=[rows2(hid), rows2(d_out)],
        out_shape=[jax.ShapeDtypeStruct((n, hid), jnp.float32),
                   jax.ShapeDtypeStruct((n, d_out), jnp.float32)],
    )(s1p, cntp, x, W1_l.T, b1_l.reshape(1, -1), W1_r.T, W2_l.T)

    s2p = _make_segsum(n, d_out, nch)(g, src, dst)

    out = pl.pallas_call(
        _tc2_body,
        grid=grid,
        in_specs=[rows3(d_out), rows3(d_in), rows2(hid),
                  full((hid, d_out)), full((1, d_out))],
        out_specs=rows2(d_out),
        out_shape=jax.ShapeDtypeStruct((n, d_out), jnp.float32),
    )(s2p, cntp, h, W2_r.T, b2_l.reshape(1, -1))
    return out


# speedup vs baseline: 6.6036x; 1.3228x over previous
import jax
import jax.numpy as jnp
from jax import lax
from jax.experimental import pallas as pl
from jax.experimental.pallas import tpu as pltpu
from jax.experimental.pallas import tpu_sc as plsc

NC = 2
NS = 16
NW = NC * NS
C = 128
CW = 16
ZR = 24


def _make_segsum(n, d, nch):
    epw = nch * C
    rw = (n // NS) // 8 * 8
    rem = n - NS * rw
    np1 = n + 1
    zr = 2 * C
    mesh = plsc.VectorSubcoreMesh(core_axis_name="c", subcore_axis_name="s",
                                  num_cores=NC, num_subcores=NS)
    scratch = [
        pltpu.VMEM_SHARED((np1, d), jnp.float32),
        pltpu.VMEM((4, C), jnp.int32),
        pltpu.VMEM((4, C), jnp.int32),
        pltpu.VMEM((2 * C, d), jnp.float32),
        pltpu.SemaphoreType.DMA,
        pltpu.SemaphoreType.DMA,
    ]

    def body(vals_hbm, src_hbm, dst_hbm, out_hbm,
             acc_sh, srcc, dstc, rows_v, gsem, isem):
        cid = lax.axis_index("c")
        sid = lax.axis_index("s")
        wid = sid * NC + cid

        z16 = jnp.zeros((16,), jnp.float32)
        dl = d // 16

        def zb(i, _):
            rows_v[i // dl, pl.ds((i % dl) * 16, 16)] = z16
            return 0
        lax.fori_loop(0, zr * dl, zb, 0)
        for t in range(rw // zr):
            pltpu.sync_copy(rows_v, acc_sh.at[pl.ds(sid * rw + t * zr, zr)])
        rz = rw - (rw // zr) * zr
        if rz:
            pltpu.sync_copy(rows_v.at[pl.ds(0, rz)],
                            acc_sh.at[pl.ds(sid * rw + rw - rz, rz)])

        @pl.when(sid == NS - 1)
        def _():
            pltpu.sync_copy(rows_v.at[pl.ds(0, rem + 1)],
                            acc_sh.at[pl.ds(NS * rw, rem + 1)])

        plsc.subcore_barrier()

        base = wid * epw

        def idx_copies(j):
            return (pltpu.make_async_copy(
                        src_hbm.at[pl.ds(base + j * C, C)], srcc.at[j % 4],
                        isem),
                    pltpu.make_async_copy(
                        dst_hbm.at[pl.ds(base + j * C, C)], dstc.at[j % 4],
                        isem))

        def gather(j):
            return pltpu.make_async_copy(
                vals_hbm.at[srcc.at[j % 4]],
                rows_v.at[pl.ds((j % 2) * C, C)], gsem)

        def scatter(j):
            pltpu.sync_copy(rows_v.at[pl.ds((j % 2) * C, C)],
                            acc_sh.at[dstc.at[j % 4]], add=True)

        for cp in idx_copies(0):
            cp.start()
        for cp in idx_copies(1):
            cp.start()

        def step(j, _):
            for cp in idx_copies(j):
                cp.wait()
            gather(j).start()

            @pl.when(j > 0)
            def _():
                gather(j - 1).wait()
                scatter(j - 1)

            @pl.when(j + 2 < nch)
            def _():
                for cp in idx_copies(j + 2):
                    cp.start()
            return 0
        lax.fori_loop(0, nch, step, 0)

        gather(nch - 1).wait()
        scatter(nch - 1)

        plsc.subcore_barrier()

        pltpu.sync_copy(acc_sh.at[pl.ds(sid * rw, rw)],
                        out_hbm.at[cid, pl.ds(sid * rw, rw)])

        @pl.when(sid == NS - 1)
        def _():
            pltpu.sync_copy(acc_sh.at[pl.ds(NS * rw, rem)],
                            out_hbm.at[cid, pl.ds(NS * rw, rem)])

    return pl.kernel(body,
                     out_type=jax.ShapeDtypeStruct((NC, n, d), jnp.float32),
                     mesh=mesh, scratch_types=scratch)


def _make_cnt(n, d, nch):
    epw = nch * C
    rw = (n // NS) // 8 * 8
    rem = n - NS * rw
    np1 = n + 1
    mesh = plsc.VectorSubcoreMesh(core_axis_name="c", subcore_axis_name="s",
                                  num_cores=NC, num_subcores=NS)
    scratch = [
        pltpu.VMEM_SHARED((np1, d), jnp.float32),
        pltpu.VMEM((4, C), jnp.int32),
        pltpu.VMEM((C, d), jnp.float32),
        pltpu.VMEM((ZR, d), jnp.float32),
        pltpu.SemaphoreType.DMA,
        pltpu.SemaphoreType.DMA,
    ]

    def body(dst_hbm, outc_hbm, cnt_sh, dstc, ones_v, zcnt, csem, isem):
        cid = lax.axis_index("c")
        sid = lax.axis_index("s")
        wid = sid * NC + cid

        z16 = jnp.zeros((16,), jnp.float32)
        o16 = jnp.ones((16,), jnp.float32)
        dl = d // 16

        def zc(i, _):
            zcnt[i // dl, pl.ds((i % dl) * 16, 16)] = z16
            return 0
        lax.fori_loop(0, ZR * dl, zc, 0)

        def ob(i, _):
            ones_v[i // dl, pl.ds((i % dl) * 16, 16)] = o16
            return 0
        lax.fori_loop(0, C * dl, ob, 0)
        for t in range(rw // ZR):
            pltpu.sync_copy(zcnt, cnt_sh.at[pl.ds(sid * rw + t * ZR, ZR)])

        @pl.when(sid == NS - 1)
        def _():
            pltpu.sync_copy(zcnt.at[pl.ds(0, rem + 1)],
                            cnt_sh.at[pl.ds(NS * rw, rem + 1)])

        plsc.subcore_barrier()

        base = wid * epw

        def idx_copy(j):
            return pltpu.make_async_copy(
                dst_hbm.at[pl.ds(base + j * C, C)], dstc.at[j % 4], isem)

        def scat(j):
            return pltpu.make_async_copy(
                ones_v, cnt_sh.at[dstc.at[j % 4]], csem)

        idx_copy(0).start()
        idx_copy(1).start()

        def step(j, _):
            idx_copy(j).wait()
            pltpu.async_copy(ones_v, cnt_sh.at[dstc.at[j % 4]], csem,
                             add=True)

            @pl.when(j > 0)
            def _():
                scat(j - 1).wait()

            @pl.when(j + 2 < nch)
            def _():
                idx_copy(j + 2).start()
            return 0
        lax.fori_loop(0, nch, step, 0)

        scat(nch - 1).wait()

        plsc.subcore_barrier()

        pltpu.sync_copy(cnt_sh.at[pl.ds(sid * rw, rw)],
                        outc_hbm.at[cid, pl.ds(sid * rw, rw)])

        @pl.when(sid == NS - 1)
        def _():
            pltpu.sync_copy(cnt_sh.at[pl.ds(NS * rw, rem)],
                            outc_hbm.at[cid, pl.ds(NS * rw, rem)])

    return pl.kernel(body,
                     out_type=jax.ShapeDtypeStruct((NC, n, d), jnp.float32),
                     mesh=mesh, scratch_types=scratch)


def _tc1_body(s1p, cntp, x, w1lT, b1, w1rT, w2lT, h_out, g_out):
    s1 = s1p[0] + s1p[1]
    c = cntp[0] + cntp[1]
    inv = 1.0 / jnp.maximum(c[:, 0:1], 1.0)
    t = (jnp.dot(s1 * inv, w1lT[...], preferred_element_type=jnp.float32)
         + jnp.dot(x[...], w1rT[...], preferred_element_type=jnp.float32)
         + b1[...])
    h = jnp.maximum(t, 0.0)
    h_out[...] = h
    g_out[...] = jnp.dot(h, w2lT[...], preferred_element_type=jnp.float32)


def _tc2_body(s2p, cntp, h, w2rT, b2, out):
    s2 = s2p[0] + s2p[1]
    c = cntp[0] + cntp[1]
    inv = 1.0 / jnp.maximum(c[:, 0:1], 1.0)
    out[...] = (s2 * inv
                + jnp.dot(h[...], w2rT[...], preferred_element_type=jnp.float32)
                + b2[...])


def kernel(x, ei, W1_l, b1_l, W1_r, W2_l, b2_l, W2_r):
    n, d_in = x.shape
    e = ei.shape[1]
    hid = W1_l.shape[0]
    d_out = W2_l.shape[0]

    epc = NW * C
    nch = -(-e // epc)
    pad = nch * epc - e
    src = jnp.concatenate([ei[0], jnp.zeros((pad,), jnp.int32)])
    dst = jnp.concatenate([ei[1], jnp.full((pad,), n, jnp.int32)])

    cntp = _make_cnt(n, d_in, nch)(dst)
    s1p = _make_segsum(n, d_in, nch)(x, src, dst)

    blk = 2000
    grid = (n // blk,)
    full = lambda shape: pl.BlockSpec(shape, lambda i: tuple(0 for _ in shape))
    rows3 = lambda w: pl.BlockSpec((NC, blk, w), lambda i: (0, i, 0))
    rows2 = lambda w: pl.BlockSpec((blk, w), lambda i: (i, 0))

    h, g = pl.pallas_call(
        _tc1_body,
        grid=grid,
        in_specs=[rows3(d_in), rows3(d_in), rows2(d_in),
                  full((d_in, hid)), full((1, hid)), full((d_in, hid)),
                  full((hid, d_out))],
        out_specs=[rows2(hid), rows2(d_out)],
        out_shape=[jax.ShapeDtypeStruct((n, hid), jnp.float32),
                   jax.ShapeDtypeStruct((n, d_out), jnp.float32)],
    )(s1p, cntp, x, W1_l.T, b1_l.reshape(1, -1), W1_r.T, W2_l.T)

    s2p = _make_segsum(n, d_out, nch)(g, src, dst)

    out = pl.pallas_call(
        _tc2_body,
        grid=grid,
        in_specs=[rows3(d_out), rows3(d_in), rows2(hid),
                  full((hid, d_out)), full((1, d_out))],
        out_specs=rows2(d_out),
        out_shape=jax.ShapeDtypeStruct((n, d_out), jnp.float32),
    )(s2p, cntp, h, W2_r.T, b2_l.reshape(1, -1))
    return out


# speedup vs baseline: 13.5924x; 2.0583x over previous
import jax
import jax.numpy as jnp
from jax import lax
from jax.experimental import pallas as pl
from jax.experimental.pallas import tpu as pltpu
from jax.experimental.pallas import tpu_sc as plsc

NC = 2
NS = 16
NW = NC * NS
C = 128
CS = 80
ZR = 24


def _make_segsum(n, d, nch):
    epw = nch * CS
    rw = (n // NS) // 8 * 8
    rem = n - NS * rw
    np1 = n + 1
    zr = 3 * CS
    mesh = plsc.VectorSubcoreMesh(core_axis_name="c", subcore_axis_name="s",
                                  num_cores=NC, num_subcores=NS)
    scratch = [
        pltpu.VMEM_SHARED((np1, d), jnp.float32),
        pltpu.VMEM((8, CS), jnp.int32),
        pltpu.VMEM((8, CS), jnp.int32),
        pltpu.VMEM((3 * CS, d), jnp.float32),
        pltpu.SemaphoreType.DMA,
        pltpu.SemaphoreType.DMA,
    ]

    def body(vals_hbm, src_hbm, dst_hbm, out_hbm,
             acc_sh, srcc, dstc, rows_v, gsem, isem):
        cid = lax.axis_index("c")
        sid = lax.axis_index("s")
        wid = sid * NC + cid

        z16 = jnp.zeros((16,), jnp.float32)
        dl = d // 16

        def zb(i, _):
            rows_v[i // dl, pl.ds((i % dl) * 16, 16)] = z16
            return 0
        lax.fori_loop(0, zr * dl, zb, 0)
        for t in range(rw // zr):
            pltpu.sync_copy(rows_v, acc_sh.at[pl.ds(sid * rw + t * zr, zr)])
        rz = rw - (rw // zr) * zr
        if rz:
            pltpu.sync_copy(rows_v.at[pl.ds(0, rz)],
                            acc_sh.at[pl.ds(sid * rw + rw - rz, rz)])

        @pl.when(sid == NS - 1)
        def _():
            pltpu.sync_copy(rows_v.at[pl.ds(0, rem + 1)],
                            acc_sh.at[pl.ds(NS * rw, rem + 1)])

        plsc.subcore_barrier()

        base = wid * epw

        def idx_copies(j):
            return (pltpu.make_async_copy(
                        src_hbm.at[pl.ds(base + j * CS, CS)], srcc.at[j % 8],
                        isem),
                    pltpu.make_async_copy(
                        dst_hbm.at[pl.ds(base + j * CS, CS)], dstc.at[j % 8],
                        isem))

        def gather(j):
            return pltpu.make_async_copy(
                vals_hbm.at[srcc.at[j % 8]],
                rows_v.at[pl.ds((j % 3) * CS, CS)], gsem)

        def scatter(j):
            pltpu.sync_copy(rows_v.at[pl.ds((j % 3) * CS, CS)],
                            acc_sh.at[dstc.at[j % 8]], add=True)

        for p in range(4):
            if p < nch:
                for cp in idx_copies(p):
                    cp.start()

        def step(j, _):
            for cp in idx_copies(j):
                cp.wait()
            gather(j).start()

            @pl.when(j > 1)
            def _():
                gather(j - 2).wait()
                scatter(j - 2)

            @pl.when(j + 4 < nch)
            def _():
                for cp in idx_copies(j + 4):
                    cp.start()
            return 0
        lax.fori_loop(0, nch, step, 0)

        if nch > 1:
            gather(nch - 2).wait()
            scatter(nch - 2)
        gather(nch - 1).wait()
        scatter(nch - 1)

        plsc.subcore_barrier()

        pltpu.sync_copy(acc_sh.at[pl.ds(sid * rw, rw)],
                        out_hbm.at[cid, pl.ds(sid * rw, rw)])

        @pl.when(sid == NS - 1)
        def _():
            pltpu.sync_copy(acc_sh.at[pl.ds(NS * rw, rem)],
                            out_hbm.at[cid, pl.ds(NS * rw, rem)])

    return pl.kernel(body,
                     out_type=jax.ShapeDtypeStruct((NC, n, d), jnp.float32),
                     mesh=mesh, scratch_types=scratch)


def _make_cnt(n, d, nch):
    epw = nch * C
    rw = (n // NS) // 8 * 8
    rem = n - NS * rw
    np1 = n + 1
    mesh = plsc.VectorSubcoreMesh(core_axis_name="c", subcore_axis_name="s",
                                  num_cores=NC, num_subcores=NS)
    scratch = [
        pltpu.VMEM_SHARED((np1, d), jnp.float32),
        pltpu.VMEM((4, C), jnp.int32),
        pltpu.VMEM((C, d), jnp.float32),
        pltpu.VMEM((ZR, d), jnp.float32),
        pltpu.SemaphoreType.DMA,
        pltpu.SemaphoreType.DMA,
    ]

    def body(dst_hbm, outc_hbm, cnt_sh, dstc, ones_v, zcnt, csem, isem):
        cid = lax.axis_index("c")
        sid = lax.axis_index("s")
        wid = sid * NC + cid

        z16 = jnp.zeros((16,), jnp.float32)
        o16 = jnp.ones((16,), jnp.float32)
        dl = d // 16

        def zc(i, _):
            zcnt[i // dl, pl.ds((i % dl) * 16, 16)] = z16
            return 0
        lax.fori_loop(0, ZR * dl, zc, 0)

        def ob(i, _):
            ones_v[i // dl, pl.ds((i % dl) * 16, 16)] = o16
            return 0
        lax.fori_loop(0, C * dl, ob, 0)
        for t in range(rw // ZR):
            pltpu.sync_copy(zcnt, cnt_sh.at[pl.ds(sid * rw + t * ZR, ZR)])

        @pl.when(sid == NS - 1)
        def _():
            pltpu.sync_copy(zcnt.at[pl.ds(0, rem + 1)],
                            cnt_sh.at[pl.ds(NS * rw, rem + 1)])

        plsc.subcore_barrier()

        base = wid * epw

        def idx_copy(j):
            return pltpu.make_async_copy(
                dst_hbm.at[pl.ds(base + j * C, C)], dstc.at[j % 4], isem)

        def scat(j):
            return pltpu.make_async_copy(
                ones_v, cnt_sh.at[dstc.at[j % 4]], csem)

        idx_copy(0).start()
        idx_copy(1).start()

        def step(j, _):
            idx_copy(j).wait()
            pltpu.async_copy(ones_v, cnt_sh.at[dstc.at[j % 4]], csem,
                             add=True)

            @pl.when(j > 0)
            def _():
                scat(j - 1).wait()

            @pl.when(j + 2 < nch)
            def _():
                idx_copy(j + 2).start()
            return 0
        lax.fori_loop(0, nch, step, 0)

        scat(nch - 1).wait()

        plsc.subcore_barrier()

        pltpu.sync_copy(cnt_sh.at[pl.ds(sid * rw, rw)],
                        outc_hbm.at[cid, pl.ds(sid * rw, rw)])

        @pl.when(sid == NS - 1)
        def _():
            pltpu.sync_copy(cnt_sh.at[pl.ds(NS * rw, rem)],
                            outc_hbm.at[cid, pl.ds(NS * rw, rem)])

    return pl.kernel(body,
                     out_type=jax.ShapeDtypeStruct((NC, n, d), jnp.float32),
                     mesh=mesh, scratch_types=scratch)


def _tc1_body(s1p, cntp, x, w1lT, b1, w1rT, w2lT, h_out, g_out):
    s1 = s1p[0] + s1p[1]
    c = cntp[0] + cntp[1]
    inv = 1.0 / jnp.maximum(c[:, 0:1], 1.0)
    t = (jnp.dot(s1 * inv, w1lT[...], preferred_element_type=jnp.float32)
         + jnp.dot(x[...], w1rT[...], preferred_element_type=jnp.float32)
         + b1[...])
    h = jnp.maximum(t, 0.0)
    h_out[...] = h
    g_out[...] = jnp.dot(h, w2lT[...], preferred_element_type=jnp.float32)


def _tc2_body(s2p, cntp, h, w2rT, b2, out):
    s2 = s2p[0] + s2p[1]
    c = cntp[0] + cntp[1]
    inv = 1.0 / jnp.maximum(c[:, 0:1], 1.0)
    out[...] = (s2 * inv
                + jnp.dot(h[...], w2rT[...], preferred_element_type=jnp.float32)
                + b2[...])


def kernel(x, ei, W1_l, b1_l, W1_r, W2_l, b2_l, W2_r):
    n, d_in = x.shape
    e = ei.shape[1]
    hid = W1_l.shape[0]
    d_out = W2_l.shape[0]

    epc_c = NW * C
    nch_c = -(-e // epc_c)
    pad_c = nch_c * epc_c - e
    dst_c = jnp.concatenate([ei[1], jnp.full((pad_c,), n, jnp.int32)])

    epc_s = NW * CS
    nch_s = -(-e // epc_s)
    pad_s = nch_s * epc_s - e
    src = jnp.concatenate([ei[0], jnp.zeros((pad_s,), jnp.int32)])
    dst = jnp.concatenate([ei[1], jnp.full((pad_s,), n, jnp.int32)])

    cntp = _make_cnt(n, d_in, nch_c)(dst_c)
    s1p = _make_segsum(n, d_in, nch_s)(x, src, dst)

    blk = 2000
    grid = (n // blk,)
    full = lambda shape: pl.BlockSpec(shape, lambda i: tuple(0 for _ in shape))
    rows3 = lambda w: pl.BlockSpec((NC, blk, w), lambda i: (0, i, 0))
    rows2 = lambda w: pl.BlockSpec((blk, w), lambda i: (i, 0))

    h, g = pl.pallas_call(
        _tc1_body,
        grid=grid,
        in_specs=[rows3(d_in), rows3(d_in), rows2(d_in),
                  full((d_in, hid)), full((1, hid)), full((d_in, hid)),
                  full((hid, d_out))],
        out_specs=[rows2(hid), rows2(d_out)],
        out_shape=[jax.ShapeDtypeStruct((n, hid), jnp.float32),
                   jax.ShapeDtypeStruct((n, d_out), jnp.float32)],
    )(s1p, cntp, x, W1_l.T, b1_l.reshape(1, -1), W1_r.T, W2_l.T)

    s2p = _make_segsum(n, d_out, nch_s)(g, src, dst)

    out = pl.pallas_call(
        _tc2_body,
        grid=grid,
        in_specs=[rows3(d_out), rows3(d_in), rows2(hid),
                  full((hid, d_out)), full((1, d_out))],
        out_specs=rows2(d_out),
        out_shape=jax.ShapeDtypeStruct((n, d_out), jnp.float32),
    )(s2p, cntp, h, W2_r.T, b2_l.reshape(1, -1))
    return out


# speedup vs baseline: 13.6993x; 1.0079x over previous
import jax
import jax.numpy as jnp
from jax import lax
from jax.experimental import pallas as pl
from jax.experimental.pallas import tpu as pltpu
from jax.experimental.pallas import tpu_sc as plsc

NC = 2
NS = 16
NW = NC * NS
C = 128
CS = 80
ZR = 24


def _make_segsum(n, d, nch):
    epw = nch * CS
    rw = (n // NS) // 8 * 8
    rem = n - NS * rw
    np1 = n + 1
    zr = 3 * CS
    mesh = plsc.VectorSubcoreMesh(core_axis_name="c", subcore_axis_name="s",
                                  num_cores=NC, num_subcores=NS)
    scratch = [
        pltpu.VMEM_SHARED((np1, d), jnp.float32),
        pltpu.VMEM((8, CS), jnp.int32),
        pltpu.VMEM((8, CS), jnp.int32),
        pltpu.VMEM((3 * CS, d), jnp.float32),
        pltpu.SemaphoreType.DMA,
        pltpu.SemaphoreType.DMA,
    ]

    def body(vals_hbm, src_hbm, dst_hbm, out_hbm,
             acc_sh, srcc, dstc, rows_v, gsem, isem):
        cid = lax.axis_index("c")
        sid = lax.axis_index("s")
        wid = sid * NC + cid

        z16 = jnp.zeros((16,), jnp.float32)
        dl = d // 16

        def zb(i, _):
            rows_v[i // dl, pl.ds((i % dl) * 16, 16)] = z16
            return 0
        lax.fori_loop(0, zr * dl, zb, 0)
        for t in range(rw // zr):
            pltpu.sync_copy(rows_v, acc_sh.at[pl.ds(sid * rw + t * zr, zr)])
        rz = rw - (rw // zr) * zr
        if rz:
            pltpu.sync_copy(rows_v.at[pl.ds(0, rz)],
                            acc_sh.at[pl.ds(sid * rw + rw - rz, rz)])

        @pl.when(sid == NS - 1)
        def _():
            pltpu.sync_copy(rows_v.at[pl.ds(0, rem + 1)],
                            acc_sh.at[pl.ds(NS * rw, rem + 1)])

        plsc.subcore_barrier()

        base = wid * epw

        def idx_copies(j):
            return (pltpu.make_async_copy(
                        src_hbm.at[pl.ds(base + j * CS, CS)], srcc.at[j % 8],
                        isem),
                    pltpu.make_async_copy(
                        dst_hbm.at[pl.ds(base + j * CS, CS)], dstc.at[j % 8],
                        isem))

        def gather(j):
            return pltpu.make_async_copy(
                vals_hbm.at[srcc.at[j % 8]],
                rows_v.at[pl.ds((j % 3) * CS, CS)], gsem)

        def scatter(j):
            pltpu.sync_copy(rows_v.at[pl.ds((j % 3) * CS, CS)],
                            acc_sh.at[dstc.at[j % 8]], add=True)

        for p in range(4):
            if p < nch:
                for cp in idx_copies(p):
                    cp.start()

        def step(j, _):
            for cp in idx_copies(j):
                cp.wait()
            gather(j).start()

            @pl.when(j > 1)
            def _():
                gather(j - 2).wait()
                scatter(j - 2)

            @pl.when(j + 4 < nch)
            def _():
                for cp in idx_copies(j + 4):
                    cp.start()
            return 0
        lax.fori_loop(0, nch, step, 0)

        if nch > 1:
            gather(nch - 2).wait()
            scatter(nch - 2)
        gather(nch - 1).wait()
        scatter(nch - 1)

        plsc.subcore_barrier()

        pltpu.sync_copy(acc_sh.at[pl.ds(sid * rw, rw)],
                        out_hbm.at[cid, pl.ds(sid * rw, rw)])

        @pl.when(sid == NS - 1)
        def _():
            pltpu.sync_copy(acc_sh.at[pl.ds(NS * rw, rem)],
                            out_hbm.at[cid, pl.ds(NS * rw, rem)])

    return pl.kernel(body,
                     out_type=jax.ShapeDtypeStruct((NC, n, d), jnp.float32),
                     mesh=mesh, scratch_types=scratch)


def _make_cnt(n, d, nch):
    epw = nch * CS
    rw = (n // NS) // 8 * 8
    rem = n - NS * rw
    np1 = n + 1
    mesh = plsc.VectorSubcoreMesh(core_axis_name="c", subcore_axis_name="s",
                                  num_cores=NC, num_subcores=NS)
    scratch = [
        pltpu.VMEM_SHARED((np1, d), jnp.float32),
        pltpu.VMEM((4, CS), jnp.int32),
        pltpu.VMEM((CS, d), jnp.float32),
        pltpu.VMEM((ZR, d), jnp.float32),
        pltpu.SemaphoreType.DMA,
        pltpu.SemaphoreType.DMA,
    ]

    def body(dst_hbm, outc_hbm, cnt_sh, dstc, ones_v, zcnt, csem, isem):
        cid = lax.axis_index("c")
        sid = lax.axis_index("s")
        wid = sid * NC + cid

        z16 = jnp.zeros((16,), jnp.float32)
        o16 = jnp.ones((16,), jnp.float32)
        dl = d // 16

        def zc(i, _):
            zcnt[i // dl, pl.ds((i % dl) * 16, 16)] = z16
            return 0
        lax.fori_loop(0, ZR * dl, zc, 0)

        def ob(i, _):
            ones_v[i // dl, pl.ds((i % dl) * 16, 16)] = o16
            return 0
        lax.fori_loop(0, CS * dl, ob, 0)
        for t in range(rw // ZR):
            pltpu.sync_copy(zcnt, cnt_sh.at[pl.ds(sid * rw + t * ZR, ZR)])

        @pl.when(sid == NS - 1)
        def _():
            pltpu.sync_copy(zcnt.at[pl.ds(0, rem + 1)],
                            cnt_sh.at[pl.ds(NS * rw, rem + 1)])

        plsc.subcore_barrier()

        base = wid * epw

        def idx_copy(j):
            return pltpu.make_async_copy(
                dst_hbm.at[pl.ds(base + j * CS, CS)], dstc.at[j % 4], isem)

        def scat(j):
            return pltpu.make_async_copy(
                ones_v, cnt_sh.at[dstc.at[j % 4]], csem)

        idx_copy(0).start()
        idx_copy(1).start()

        def step(j, _):
            idx_copy(j).wait()
            pltpu.async_copy(ones_v, cnt_sh.at[dstc.at[j % 4]], csem,
                             add=True)

            @pl.when(j > 0)
            def _():
                scat(j - 1).wait()

            @pl.when(j + 2 < nch)
            def _():
                idx_copy(j + 2).start()
            return 0
        lax.fori_loop(0, nch, step, 0)

        scat(nch - 1).wait()

        plsc.subcore_barrier()

        pltpu.sync_copy(cnt_sh.at[pl.ds(sid * rw, rw)],
                        outc_hbm.at[cid, pl.ds(sid * rw, rw)])

        @pl.when(sid == NS - 1)
        def _():
            pltpu.sync_copy(cnt_sh.at[pl.ds(NS * rw, rem)],
                            outc_hbm.at[cid, pl.ds(NS * rw, rem)])

    return pl.kernel(body,
                     out_type=jax.ShapeDtypeStruct((NC, n, d), jnp.float32),
                     mesh=mesh, scratch_types=scratch)


def _tc1_body(s1p, cntp, x, w1lT, b1, w1rT, w2lT, h_out, g_out):
    s1 = s1p[0] + s1p[1]
    c = cntp[0] + cntp[1]
    inv = 1.0 / jnp.maximum(c[:, 0:1], 1.0)
    t = (jnp.dot(s1 * inv, w1lT[...], preferred_element_type=jnp.float32)
         + jnp.dot(x[...], w1rT[...], preferred_element_type=jnp.float32)
         + b1[...])
    h = jnp.maximum(t, 0.0)
    h_out[...] = h
    g_out[...] = jnp.dot(h, w2lT[...], preferred_element_type=jnp.float32)


def _tc2_body(s2p, cntp, h, w2rT, b2, out):
    s2 = s2p[0] + s2p[1]
    c = cntp[0] + cntp[1]
    inv = 1.0 / jnp.maximum(c[:, 0:1], 1.0)
    out[...] = (s2 * inv
                + jnp.dot(h[...], w2rT[...], preferred_element_type=jnp.float32)
                + b2[...])


def kernel(x, ei, W1_l, b1_l, W1_r, W2_l, b2_l, W2_r):
    n, d_in = x.shape
    e = ei.shape[1]
    hid = W1_l.shape[0]
    d_out = W2_l.shape[0]

    epc_s = NW * CS
    nch_s = -(-e // epc_s)
    pad_s = nch_s * epc_s - e
    if pad_s:
        src = jnp.concatenate([ei[0], jnp.zeros((pad_s,), jnp.int32)])
        dst = jnp.concatenate([ei[1], jnp.full((pad_s,), n, jnp.int32)])
    else:
        src, dst = ei[0], ei[1]

    cntp = _make_cnt(n, d_in, nch_s)(dst)
    s1p = _make_segsum(n, d_in, nch_s)(x, src, dst)

    blk = 2000
    grid = (n // blk,)
    full = lambda shape: pl.BlockSpec(shape, lambda i: tuple(0 for _ in shape))
    rows3 = lambda w: pl.BlockSpec((NC, blk, w), lambda i: (0, i, 0))
    rows2 = lambda w: pl.BlockSpec((blk, w), lambda i: (i, 0))

    h, g = pl.pallas_call(
        _tc1_body,
        grid=grid,
        in_specs=[rows3(d_in), rows3(d_in), rows2(d_in),
                  full((d_in, hid)), full((1, hid)), full((d_in, hid)),
                  full((hid, d_out))],
        out_specs=[rows2(hid), rows2(d_out)],
        out_shape=[jax.ShapeDtypeStruct((n, hid), jnp.float32),
                   jax.ShapeDtypeStruct((n, d_out), jnp.float32)],
    )(s1p, cntp, x, W1_l.T, b1_l.reshape(1, -1), W1_r.T, W2_l.T)

    s2p = _make_segsum(n, d_out, nch_s)(g, src, dst)

    out = pl.pallas_call(
        _tc2_body,
        grid=grid,
        in_specs=[rows3(d_out), rows3(d_in), rows2(hid),
                  full((hid, d_out)), full((1, d_out))],
        out_specs=rows2(d_out),
        out_shape=jax.ShapeDtypeStruct((n, d_out), jnp.float32),
    )(s2p, cntp, h, W2_r.T, b2_l.reshape(1, -1))
    return out


# speedup vs baseline: 13.7015x; 1.0002x over previous
import jax
import jax.numpy as jnp
from jax import lax
from jax.experimental import pallas as pl
from jax.experimental.pallas import tpu as pltpu
from jax.experimental.pallas import tpu_sc as plsc

NC = 2
NS = 16
NW = NC * NS
C = 128
CS = 80
ZR = 24


def _make_segsum(n, d, nch):
    epw = nch * CS
    rw = (n // NS) // 8 * 8
    rem = n - NS * rw
    np1 = n + 1
    zr = 3 * CS
    mesh = plsc.VectorSubcoreMesh(core_axis_name="c", subcore_axis_name="s",
                                  num_cores=NC, num_subcores=NS)
    scratch = [
        pltpu.VMEM_SHARED((np1, d), jnp.float32),
        pltpu.VMEM((8, CS), jnp.int32),
        pltpu.VMEM((8, CS), jnp.int32),
        pltpu.VMEM((3 * CS, d), jnp.float32),
        pltpu.SemaphoreType.DMA,
        pltpu.SemaphoreType.DMA,
    ]

    def body(vals_hbm, src_hbm, dst_hbm, out_hbm,
             acc_sh, srcc, dstc, rows_v, gsem, isem):
        cid = lax.axis_index("c")
        sid = lax.axis_index("s")
        wid = sid * NC + cid

        z16 = jnp.zeros((16,), jnp.float32)
        dl = d // 16

        def zb(i, _):
            rows_v[i // dl, pl.ds((i % dl) * 16, 16)] = z16
            return 0
        lax.fori_loop(0, zr * dl, zb, 0)
        for t in range(rw // zr):
            pltpu.sync_copy(rows_v, acc_sh.at[pl.ds(sid * rw + t * zr, zr)])
        rz = rw - (rw // zr) * zr
        if rz:
            pltpu.sync_copy(rows_v.at[pl.ds(0, rz)],
                            acc_sh.at[pl.ds(sid * rw + rw - rz, rz)])

        @pl.when(sid == NS - 1)
        def _():
            pltpu.sync_copy(rows_v.at[pl.ds(0, rem + 1)],
                            acc_sh.at[pl.ds(NS * rw, rem + 1)])

        plsc.subcore_barrier()

        base = wid * epw

        def idx_copies(j):
            return (pltpu.make_async_copy(
                        src_hbm.at[pl.ds(base + j * CS, CS)], srcc.at[j % 8],
                        isem),
                    pltpu.make_async_copy(
                        dst_hbm.at[pl.ds(base + j * CS, CS)], dstc.at[j % 8],
                        isem))

        def gather(j):
            return pltpu.make_async_copy(
                vals_hbm.at[srcc.at[j % 8]],
                rows_v.at[pl.ds((j % 3) * CS, CS)], gsem)

        def scatter(j):
            pltpu.sync_copy(rows_v.at[pl.ds((j % 3) * CS, CS)],
                            acc_sh.at[dstc.at[j % 8]], add=True)

        for p in range(4):
            if p < nch:
                for cp in idx_copies(p):
                    cp.start()

        def step(j, _):
            for cp in idx_copies(j):
                cp.wait()
            gather(j).start()

            @pl.when(j > 1)
            def _():
                gather(j - 2).wait()
                scatter(j - 2)

            @pl.when(j + 4 < nch)
            def _():
                for cp in idx_copies(j + 4):
                    cp.start()
            return 0
        lax.fori_loop(0, nch, step, 0)

        if nch > 1:
            gather(nch - 2).wait()
            scatter(nch - 2)
        gather(nch - 1).wait()
        scatter(nch - 1)

        plsc.subcore_barrier()

        pltpu.sync_copy(acc_sh.at[pl.ds(sid * rw, rw)],
                        out_hbm.at[cid, pl.ds(sid * rw, rw)])

        @pl.when(sid == NS - 1)
        def _():
            pltpu.sync_copy(acc_sh.at[pl.ds(NS * rw, rem)],
                            out_hbm.at[cid, pl.ds(NS * rw, rem)])

    return pl.kernel(body,
                     out_type=jax.ShapeDtypeStruct((NC, n, d), jnp.float32),
                     mesh=mesh, scratch_types=scratch)


def _make_cnt(n, d, nch):
    epw = nch * CS
    rw = (n // NS) // 8 * 8
    rem = n - NS * rw
    np1 = n + 1
    mesh = plsc.VectorSubcoreMesh(core_axis_name="c", subcore_axis_name="s",
                                  num_cores=NC, num_subcores=NS)
    scratch = [
        pltpu.VMEM_SHARED((np1, d), jnp.float32),
        pltpu.VMEM((4, CS), jnp.int32),
        pltpu.VMEM((CS, d), jnp.float32),
        pltpu.VMEM((ZR, d), jnp.float32),
        pltpu.SemaphoreType.DMA,
        pltpu.SemaphoreType.DMA,
    ]

    def body(dst_hbm, outc_hbm, cnt_sh, dstc, ones_v, zcnt, csem, isem):
        cid = lax.axis_index("c")
        sid = lax.axis_index("s")
        wid = sid * NC + cid

        z16 = jnp.zeros((16,), jnp.float32)
        o16 = jnp.ones((16,), jnp.float32)
        dl = d // 16

        def zc(i, _):
            zcnt[i // dl, pl.ds((i % dl) * 16, 16)] = z16
            return 0
        lax.fori_loop(0, ZR * dl, zc, 0)

        def ob(i, _):
            ones_v[i // dl, pl.ds((i % dl) * 16, 16)] = o16
            return 0
        lax.fori_loop(0, CS * dl, ob, 0)
        for t in range(rw // ZR):
            pltpu.sync_copy(zcnt, cnt_sh.at[pl.ds(sid * rw + t * ZR, ZR)])

        @pl.when(sid == NS - 1)
        def _():
            pltpu.sync_copy(zcnt.at[pl.ds(0, rem + 1)],
                            cnt_sh.at[pl.ds(NS * rw, rem + 1)])

        plsc.subcore_barrier()

        base = wid * epw

        def idx_copy(j):
            return pltpu.make_async_copy(
                dst_hbm.at[pl.ds(base + j * CS, CS)], dstc.at[j % 4], isem)

        def scat(j):
            return pltpu.make_async_copy(
                ones_v, cnt_sh.at[dstc.at[j % 4]], csem)

        idx_copy(0).start()
        idx_copy(1).start()

        def step(j, _):
            idx_copy(j).wait()
            pltpu.async_copy(ones_v, cnt_sh.at[dstc.at[j % 4]], csem,
                             add=True)

            @pl.when(j > 0)
            def _():
                scat(j - 1).wait()

            @pl.when(j + 2 < nch)
            def _():
                idx_copy(j + 2).start()
            return 0
        lax.fori_loop(0, nch, step, 0)

        scat(nch - 1).wait()

        plsc.subcore_barrier()

        pltpu.sync_copy(cnt_sh.at[pl.ds(sid * rw, rw)],
                        outc_hbm.at[cid, pl.ds(sid * rw, rw)])

        @pl.when(sid == NS - 1)
        def _():
            pltpu.sync_copy(cnt_sh.at[pl.ds(NS * rw, rem)],
                            outc_hbm.at[cid, pl.ds(NS * rw, rem)])

    return pl.kernel(body,
                     out_type=jax.ShapeDtypeStruct((NC, n, d), jnp.float32),
                     mesh=mesh, scratch_types=scratch)


def _dot_t(a, w):
    return lax.dot_general(a, w, (((1,), (1,)), ((), ())),
                           preferred_element_type=jnp.float32)


def _tc1_body(s1p, cntp, x, w1l, b1, w1r, w2l, h_out, g_out):
    s1 = s1p[0] + s1p[1]
    c = cntp[0] + cntp[1]
    inv = 1.0 / jnp.maximum(c[:, 0:1], 1.0)
    t = _dot_t(s1 * inv, w1l[...]) + _dot_t(x[...], w1r[...]) + b1[...]
    h = jnp.maximum(t, 0.0)
    h_out[...] = h
    g_out[...] = _dot_t(h, w2l[...])


def _tc2_body(s2p, cntp, h, w2r, b2, out):
    s2 = s2p[0] + s2p[1]
    c = cntp[0] + cntp[1]
    inv = 1.0 / jnp.maximum(c[:, 0:1], 1.0)
    out[...] = s2 * inv + _dot_t(h[...], w2r[...]) + b2[...]


def kernel(x, ei, W1_l, b1_l, W1_r, W2_l, b2_l, W2_r):
    n, d_in = x.shape
    e = ei.shape[1]
    hid = W1_l.shape[0]
    d_out = W2_l.shape[0]

    epc_s = NW * CS
    nch_s = -(-e // epc_s)
    pad_s = nch_s * epc_s - e
    if pad_s:
        src = jnp.concatenate([ei[0], jnp.zeros((pad_s,), jnp.int32)])
        dst = jnp.concatenate([ei[1], jnp.full((pad_s,), n, jnp.int32)])
    else:
        src, dst = ei[0], ei[1]

    cntp = _make_cnt(n, d_in, nch_s)(dst)
    s1p = _make_segsum(n, d_in, nch_s)(x, src, dst)

    blk = 2000
    grid = (n // blk,)
    full = lambda shape: pl.BlockSpec(shape, lambda i: tuple(0 for _ in shape))
    rows3 = lambda w: pl.BlockSpec((NC, blk, w), lambda i: (0, i, 0))
    rows2 = lambda w: pl.BlockSpec((blk, w), lambda i: (i, 0))

    h, g = pl.pallas_call(
        _tc1_body,
        grid=grid,
        in_specs=[rows3(d_in), rows3(d_in), rows2(d_in),
                  full((hid, d_in)), full((1, hid)), full((hid, d_in)),
                  full((d_out, hid))],
        out_specs=[rows2(hid), rows2(d_out)],
        out_shape=[jax.ShapeDtypeStruct((n, hid), jnp.float32),
                   jax.ShapeDtypeStruct((n, d_out), jnp.float32)],
    )(s1p, cntp, x, W1_l, b1_l.reshape(1, -1), W1_r, W2_l)

    s2p = _make_segsum(n, d_out, nch_s)(g, src, dst)

    out = pl.pallas_call(
        _tc2_body,
        grid=grid,
        in_specs=[rows3(d_out), rows3(d_in), rows2(hid),
                  full((d_out, hid)), full((1, d_out))],
        out_specs=rows2(d_out),
        out_shape=jax.ShapeDtypeStruct((n, d_out), jnp.float32),
    )(s2p, cntp, h, W2_r, b2_l.reshape(1, -1))
    return out
